# Initial kernel scaffold; baseline (speedup 1.0000x reference)
import jax
import jax.numpy as jnp
from jax.experimental import pallas as pl


def kernel(x, W1, b1, memory, mean, std):
    raise NotImplementedError("write your pallas kernel here")


# speedup vs baseline: 2.2955x; 2.2955x over previous
import jax
import jax.numpy as jnp
from jax.experimental import pallas as pl
from jax.experimental.pallas import tpu as pltpu

IN_DIM = 16
OUT_DIM = 32
MEM_LEN = 100000
BATCH = 1024
BETA = 1.0

COLS = 128
PAD_LEN = ((MEM_LEN + COLS - 1) // COLS) * COLS
NBLK = PAD_LEN // COLS

ROW_BLK = 2000
NROW = MEM_LEN // ROW_BLK


def _enc_kernel(x_ref, w_ref, b_ref, mean_ref, std_ref, enc_ref):
    std = std_ref[:]
    zero = std == 0.0
    denom = jnp.where(zero, 1.0, std)
    new = jnp.where(zero, 0.0, (x_ref[:] - mean_ref[:]) / denom)
    h = jnp.dot(new, w_ref[:], preferred_element_type=jnp.float32) + b_ref[:]
    enc_ref[:] = jnp.tanh(h)


def _dist_kernel(enc_ref, mt_ref, loss_ref, minacc_ref):
    i = pl.program_id(0)
    mt = mt_ref[:]
    enc = enc_ref[:]
    acc = jnp.zeros((BATCH, COLS), dtype=jnp.float32)
    for d in range(OUT_DIM):
        acc = acc + jnp.abs(enc[:, d:d + 1] - mt[d:d + 1, :])
    prev = jnp.where(i == 0, jnp.float32(3e38), minacc_ref[:])
    minacc_ref[:] = jnp.minimum(prev, acc)
    @pl.when(i == NBLK - 1)
    def _():
        loss_ref[:] = jnp.min(minacc_ref[:], axis=1, keepdims=True)


def _update_kernel(mem_ref, loss_ref, enc_ref, out_ref):
    i = pl.program_id(0)
    out_ref[:] = mem_ref[:]
    @pl.when(i == NROW - 1)
    def _():
        loss = loss_ref[:]
        maskf = (loss <= BETA).astype(jnp.float32)
        sub = jax.lax.broadcasted_iota(jnp.int32, (BATCH, BATCH), 0)
        lane = jax.lax.broadcasted_iota(jnp.int32, (BATCH, BATCH), 1)
        tri = (lane <= sub).astype(jnp.float32)
        cnt = jax.lax.dot_general(
            tri, maskf, (((1,), (0,)), ((), ())),
            preferred_element_type=jnp.float32)
        cnt0 = cnt - 1.0
        q = ((cnt0 == lane.astype(jnp.float32)) & (maskf > 0.0)).astype(jnp.float32)
        compacted = jax.lax.dot_general(
            q, enc_ref[:], (((0,), (0,)), ((), ())),
            preferred_element_type=jnp.float32)
        total = jnp.sum(maskf)
        slot = jax.lax.broadcasted_iota(jnp.int32, (BATCH, 1), 0)
        written = slot.astype(jnp.float32) < total
        out_ref[0:BATCH, :] = jnp.where(written, compacted, mem_ref[0:BATCH, :])


def kernel(x, W1, b1, memory, mean, std):
    mt = jnp.concatenate(
        [memory.T, jnp.full((OUT_DIM, PAD_LEN - MEM_LEN), 3e38, jnp.float32)],
        axis=1)

    enc = pl.pallas_call(
        _enc_kernel,
        out_shape=jax.ShapeDtypeStruct((BATCH, OUT_DIM), jnp.float32),
    )(x, W1, b1.reshape(1, OUT_DIM), mean.reshape(1, IN_DIM),
      std.reshape(1, IN_DIM))

    loss2d = pl.pallas_call(
        _dist_kernel,
        grid=(NBLK,),
        in_specs=[
            pl.BlockSpec((BATCH, OUT_DIM), lambda i: (0, 0)),
            pl.BlockSpec((OUT_DIM, COLS), lambda i: (0, i)),
        ],
        out_specs=pl.BlockSpec((BATCH, 1), lambda i: (0, 0)),
        out_shape=jax.ShapeDtypeStruct((BATCH, 1), jnp.float32),
        scratch_shapes=[pltpu.VMEM((BATCH, COLS), jnp.float32)],
    )(enc, mt)

    new_memory = pl.pallas_call(
        _update_kernel,
        grid=(NROW,),
        in_specs=[
            pl.BlockSpec((ROW_BLK, OUT_DIM), lambda i: ((i + 1) % NROW, 0)),
            pl.BlockSpec((BATCH, 1), lambda i: (0, 0)),
            pl.BlockSpec((BATCH, OUT_DIM), lambda i: (0, 0)),
        ],
        out_specs=pl.BlockSpec((ROW_BLK, OUT_DIM), lambda i: ((i + 1) % NROW, 0)),
        out_shape=jax.ShapeDtypeStruct((MEM_LEN, OUT_DIM), jnp.float32),
    )(memory, loss2d, enc)

    return loss2d.reshape(BATCH), new_memory


# speedup vs baseline: 6.1057x; 2.6599x over previous
import jax
import jax.numpy as jnp
from jax.experimental import pallas as pl
from jax.experimental.pallas import tpu as pltpu

IN_DIM = 16
OUT_DIM = 32
MEM_LEN = 100000
BATCH = 1024
BETA = 1.0

COLS = 128
NBLK = (MEM_LEN + COLS - 1) // COLS
LAST_VALID = MEM_LEN - (NBLK - 1) * COLS
QC = 128
PREP_BLK = 2000
NPREP = MEM_LEN // PREP_BLK
TBLK = 2048
NT = (MEM_LEN + TBLK - 1) // TBLK
MT_LEN = NT * TBLK
FIX_BLK = 2000


def _enc_kernel(x_ref, w_ref, b_ref, mean_ref, std_ref, enc_ref):
    std = std_ref[:]
    zero = std == 0.0
    denom = jnp.where(zero, 1.0, std)
    new = jnp.where(zero, 0.0, (x_ref[:] - mean_ref[:]) / denom)
    h = jnp.dot(new, w_ref[:], preferred_element_type=jnp.float32) + b_ref[:]
    enc_ref[:] = jnp.tanh(h)


def _prep_kernel(mem_ref, memt_ref, copy_ref, mt_ref):
    copy_ref[:] = mem_ref[:]
    mt_ref[:] = memt_ref[:].T


def _dist_kernel(enc_ref, mt_ref, loss_ref, minacc_ref, ebc_ref):
    i = pl.program_id(0)
    @pl.when(i == 0)
    def _():
        for d in range(OUT_DIM):
            ebc_ref[d] = jnp.broadcast_to(enc_ref[:, d:d + 1], (BATCH, COLS))
    mt = mt_ref[:]
    lane = jax.lax.broadcasted_iota(jnp.int32, (OUT_DIM, COLS), 1)
    limit = jnp.where(i == NBLK - 1, LAST_VALID, COLS)
    mt = jnp.where(lane < limit, mt, jnp.float32(1e30))
    for qc in range(BATCH // QC):
        lo = qc * QC
        acc = jnp.zeros((QC, COLS), dtype=jnp.float32)
        for d in range(OUT_DIM):
            acc = acc + jnp.abs(ebc_ref[d, lo:lo + QC, :] - mt[d:d + 1, :])
        prev = jnp.where(i == 0, jnp.float32(3e38), minacc_ref[lo:lo + QC, :])
        minacc_ref[lo:lo + QC, :] = jnp.minimum(prev, acc)
    @pl.when(i == NBLK - 1)
    def _():
        loss_ref[:] = jnp.min(minacc_ref[:], axis=1, keepdims=True)


def _fix_kernel(copy_ref, loss_ref, enc_ref, out_ref):
    loss = loss_ref[:]
    maskf = (loss <= BETA).astype(jnp.float32)
    sub = jax.lax.broadcasted_iota(jnp.int32, (BATCH, BATCH), 0)
    lane = jax.lax.broadcasted_iota(jnp.int32, (BATCH, BATCH), 1)
    tri = (lane <= sub).astype(jnp.float32)
    cnt = jax.lax.dot_general(
        tri, maskf, (((1,), (0,)), ((), ())),
        preferred_element_type=jnp.float32)
    cnt0 = cnt - 1.0
    q = ((cnt0 == lane.astype(jnp.float32)) & (maskf > 0.0)).astype(jnp.float32)
    compacted = jax.lax.dot_general(
        q, enc_ref[:], (((0,), (0,)), ((), ())),
        preferred_element_type=jnp.float32)
    total = jnp.sum(maskf)
    slot = jax.lax.broadcasted_iota(jnp.int32, (BATCH, 1), 0)
    written = slot.astype(jnp.float32) < total
    out_ref[:] = copy_ref[:]
    out_ref[0:BATCH, :] = jnp.where(written, compacted, copy_ref[0:BATCH, :])


def kernel(x, W1, b1, memory, mean, std):
    enc = pl.pallas_call(
        _enc_kernel,
        out_shape=jax.ShapeDtypeStruct((BATCH, OUT_DIM), jnp.float32),
    )(x, W1, b1.reshape(1, OUT_DIM), mean.reshape(1, IN_DIM),
      std.reshape(1, IN_DIM))

    copy, mt = pl.pallas_call(
        _prep_kernel,
        grid=(NPREP,),
        in_specs=[
            pl.BlockSpec((PREP_BLK, OUT_DIM), lambda i: (i, 0)),
            pl.BlockSpec((TBLK, OUT_DIM), lambda i: (jnp.minimum(i, NT - 1), 0)),
        ],
        out_specs=[
            pl.BlockSpec((PREP_BLK, OUT_DIM), lambda i: (i, 0)),
            pl.BlockSpec((OUT_DIM, TBLK), lambda i: (0, jnp.minimum(i, NT - 1))),
        ],
        out_shape=[
            jax.ShapeDtypeStruct((MEM_LEN, OUT_DIM), jnp.float32),
            jax.ShapeDtypeStruct((OUT_DIM, MT_LEN), jnp.float32),
        ],
    )(memory, memory)

    loss2d = pl.pallas_call(
        _dist_kernel,
        grid=(NBLK,),
        in_specs=[
            pl.BlockSpec((BATCH, OUT_DIM), lambda i: (0, 0)),
            pl.BlockSpec((OUT_DIM, COLS), lambda i: (0, i)),
        ],
        out_specs=pl.BlockSpec((BATCH, 1), lambda i: (0, 0)),
        out_shape=jax.ShapeDtypeStruct((BATCH, 1), jnp.float32),
        scratch_shapes=[
            pltpu.VMEM((BATCH, COLS), jnp.float32),
            pltpu.VMEM((OUT_DIM, BATCH, COLS), jnp.float32),
        ],
    )(enc, mt)

    new_memory = pl.pallas_call(
        _fix_kernel,
        grid=(1,),
        in_specs=[
            pl.BlockSpec((FIX_BLK, OUT_DIM), lambda i: (0, 0)),
            pl.BlockSpec((BATCH, 1), lambda i: (0, 0)),
            pl.BlockSpec((BATCH, OUT_DIM), lambda i: (0, 0)),
        ],
        out_specs=pl.BlockSpec((FIX_BLK, OUT_DIM), lambda i: (0, 0)),
        out_shape=jax.ShapeDtypeStruct((MEM_LEN, OUT_DIM), jnp.float32),
        input_output_aliases={0: 0},
    )(copy, loss2d, enc)

    return loss2d.reshape(BATCH), new_memory


# speedup vs baseline: 6.2987x; 1.0316x over previous
import jax
import jax.numpy as jnp
from jax.experimental import pallas as pl
from jax.experimental.pallas import tpu as pltpu

IN_DIM = 16
OUT_DIM = 32
MEM_LEN = 100000
BATCH = 1024
BETA = 1.0

PREP_BLK = 2000
NPREP = MEM_LEN // PREP_BLK
TBLK = 2048
NT = (MEM_LEN + TBLK - 1) // TBLK
MT_LEN = NT * TBLK

COLS = 512
MSUB = 128
NCT = COLS // MSUB
NBLK = MT_LEN // COLS
LAST_VALID = MEM_LEN - (NBLK - 1) * COLS
QC = 128
FIX_BLK = 2000


def _enc_kernel(x_ref, w_ref, b_ref, mean_ref, std_ref, enc_ref):
    std = std_ref[:]
    zero = std == 0.0
    denom = jnp.where(zero, 1.0, std)
    new = jnp.where(zero, 0.0, (x_ref[:] - mean_ref[:]) / denom)
    h = jnp.dot(new, w_ref[:], preferred_element_type=jnp.float32) + b_ref[:]
    enc_ref[:] = jnp.tanh(h)


def _prep_kernel(mem_ref, mt_ref):
    mt_ref[:] = mem_ref[:].T


def _dist_kernel(enc_ref, mt_ref, loss_ref, copy_ref, minacc_ref, ebc_ref):
    i = pl.program_id(0)
    @pl.when(i == 0)
    def _():
        for d in range(OUT_DIM):
            ebc_ref[d] = jnp.broadcast_to(enc_ref[:, d:d + 1], (BATCH, MSUB))
    mt = mt_ref[:]
    lane = jax.lax.broadcasted_iota(jnp.int32, (OUT_DIM, COLS), 1)
    limit = jnp.where(i == NBLK - 1, LAST_VALID, COLS)
    mt = jnp.where(lane < limit, mt, jnp.float32(1e30))
    copy_ref[:] = mt_ref[:].T
    for qc in range(BATCH // QC):
        lo = qc * QC
        mcur = None
        for ct in range(NCT):
            cl = ct * MSUB
            acc = jnp.zeros((QC, MSUB), dtype=jnp.float32)
            for d in range(OUT_DIM):
                acc = acc + jnp.abs(
                    ebc_ref[d, lo:lo + QC, :] - mt[d:d + 1, cl:cl + MSUB])
            mcur = acc if ct == 0 else jnp.minimum(mcur, acc)
        prev = jnp.where(i == 0, jnp.float32(3e38), minacc_ref[lo:lo + QC, :])
        minacc_ref[lo:lo + QC, :] = jnp.minimum(prev, mcur)
    @pl.when(i == NBLK - 1)
    def _():
        loss_ref[:] = jnp.min(minacc_ref[:], axis=1, keepdims=True)


def _fix_kernel(copy_ref, loss_ref, enc_ref, out_ref):
    loss = loss_ref[:]
    maskf = (loss <= BETA).astype(jnp.float32)
    sub = jax.lax.broadcasted_iota(jnp.int32, (BATCH, BATCH), 0)
    lane = jax.lax.broadcasted_iota(jnp.int32, (BATCH, BATCH), 1)
    tri = (lane <= sub).astype(jnp.float32)
    cnt = jax.lax.dot_general(
        tri, maskf, (((1,), (0,)), ((), ())),
        preferred_element_type=jnp.float32)
    cnt0 = cnt - 1.0
    q = ((cnt0 == lane.astype(jnp.float32)) & (maskf > 0.0)).astype(jnp.float32)
    compacted = jax.lax.dot_general(
        q, enc_ref[:], (((0,), (0,)), ((), ())),
        preferred_element_type=jnp.float32)
    total = jnp.sum(maskf)
    slot = jax.lax.broadcasted_iota(jnp.int32, (BATCH, 1), 0)
    written = slot.astype(jnp.float32) < total
    out_ref[:] = copy_ref[:]
    out_ref[0:BATCH, :] = jnp.where(written, compacted, copy_ref[0:BATCH, :])


def kernel(x, W1, b1, memory, mean, std):
    enc = pl.pallas_call(
        _enc_kernel,
        out_shape=jax.ShapeDtypeStruct((BATCH, OUT_DIM), jnp.float32),
    )(x, W1, b1.reshape(1, OUT_DIM), mean.reshape(1, IN_DIM),
      std.reshape(1, IN_DIM))

    mt = pl.pallas_call(
        _prep_kernel,
        grid=(NT,),
        in_specs=[
            pl.BlockSpec((TBLK, OUT_DIM), lambda i: (i, 0)),
        ],
        out_specs=pl.BlockSpec((OUT_DIM, TBLK), lambda i: (0, i)),
        out_shape=jax.ShapeDtypeStruct((OUT_DIM, MT_LEN), jnp.float32),
    )(memory)

    loss2d, copy = pl.pallas_call(
        _dist_kernel,
        grid=(NBLK,),
        in_specs=[
            pl.BlockSpec((BATCH, OUT_DIM), lambda i: (0, 0)),
            pl.BlockSpec((OUT_DIM, COLS), lambda i: (0, i)),
        ],
        out_specs=[
            pl.BlockSpec((BATCH, 1), lambda i: (0, 0)),
            pl.BlockSpec((COLS, OUT_DIM), lambda i: (i, 0)),
        ],
        out_shape=[
            jax.ShapeDtypeStruct((BATCH, 1), jnp.float32),
            jax.ShapeDtypeStruct((MEM_LEN, OUT_DIM), jnp.float32),
        ],
        scratch_shapes=[
            pltpu.VMEM((BATCH, MSUB), jnp.float32),
            pltpu.VMEM((OUT_DIM, BATCH, MSUB), jnp.float32),
        ],
    )(enc, mt)

    new_memory = pl.pallas_call(
        _fix_kernel,
        grid=(1,),
        in_specs=[
            pl.BlockSpec((FIX_BLK, OUT_DIM), lambda i: (0, 0)),
            pl.BlockSpec((BATCH, 1), lambda i: (0, 0)),
            pl.BlockSpec((BATCH, OUT_DIM), lambda i: (0, 0)),
        ],
        out_specs=pl.BlockSpec((FIX_BLK, OUT_DIM), lambda i: (0, 0)),
        out_shape=jax.ShapeDtypeStruct((MEM_LEN, OUT_DIM), jnp.float32),
        input_output_aliases={0: 0},
    )(copy, loss2d, enc)

    return loss2d.reshape(BATCH), new_memory
